# baseline (speedup 1.0000x reference)
import jax
import jax.numpy as jnp
from jax.experimental import pallas as pl
from jax.experimental.pallas import tpu as pltpu

_B, _S, _H, _E, _F = 1, 2048, 2048, 8, 768
_FH = 384
_TT = 2048
_HT = 1024
_RT = 512


def _router_body(hs_ref, gw_ref, logits_ref, wd_ref, xb_ref):
    x = hs_ref[...]
    xb_ref[...] = x.astype(jnp.bfloat16)
    logits = jax.lax.dot_general(
        x, gw_ref[...], (((1,), (1,)), ((), ())),
        preferred_element_type=jnp.float32)
    logits_ref[...] = logits
    p = jax.nn.softmax(logits, axis=-1)
    eio = jax.lax.broadcasted_iota(jnp.int32, p.shape, 1)
    m1 = jnp.max(p, axis=-1, keepdims=True)
    i1 = jnp.argmax(p, axis=-1)[:, None]
    oh1 = eio == i1
    p2 = jnp.where(oh1, -jnp.inf, p)
    m2 = jnp.max(p2, axis=-1, keepdims=True)
    i2 = jnp.argmax(p2, axis=-1)[:, None]
    oh2 = eio == i2
    wd_ref[...] = (jnp.where(oh1, m1, 0.0)
                   + jnp.where(oh2, m2, 0.0)) / (m1 + m2)


def _expert_body(xb_ref, wd_ref, wg_ref, wu_ref, w2_ref, out_ref,
                 wgb_ref, wub_ref, w2b_ref):
    e = pl.program_id(0)
    f = pl.program_id(1)
    t = pl.program_id(2)

    @pl.when(t == 0)
    def _():
        wgb_ref[...] = wg_ref[0].astype(jnp.bfloat16)
        wub_ref[...] = wu_ref[0].astype(jnp.bfloat16)
        w2b_ref[...] = w2_ref[0].astype(jnp.bfloat16)

    @pl.when(jnp.logical_and(jnp.logical_and(e == 0, f == 0), t == 0))
    def _():
        out_ref[...] = jnp.zeros_like(out_ref)
    eio = jax.lax.broadcasted_iota(jnp.int32, (_HT, _E), 1)
    wgb = wgb_ref[...]
    wub = wub_ref[...]
    w2b = w2b_ref[...]

    row_sl = [pl.ds(t * _TT + h * _HT, _HT) for h in range(_TT // _HT)]
    xs = [xb_ref[r, :] for r in row_sl]
    gs = [jnp.dot(x, wgb, preferred_element_type=jnp.float32) for x in xs]
    us = [jnp.dot(x, wub, preferred_element_type=jnp.float32) for x in xs]
    contribs = []
    for h, r in enumerate(row_sl):
        we = jnp.sum(jnp.where(eio == e, wd_ref[r, :], 0.0),
                     axis=-1, keepdims=True)
        g, u = gs[h], us[h]
        inter = (we * (u * (g * jax.nn.sigmoid(g)))).astype(jnp.bfloat16)
        contribs.append(
            jnp.dot(inter, w2b, preferred_element_type=jnp.float32))
    for h, r in enumerate(row_sl):
        out_ref[r, :] += contribs[h]


def kernel(hidden_states, gate_w, gate_up_proj, down_proj):
    T = _B * _S
    hs = hidden_states.reshape(T, _H)
    logits, wdense, xb = pl.pallas_call(
        _router_body,
        grid=(T // _RT,),
        in_specs=[
            pl.BlockSpec((_RT, _H), lambda t: (t, 0)),
            pl.BlockSpec((_E, _H), lambda t: (0, 0)),
        ],
        out_specs=[
            pl.BlockSpec((_RT, _E), lambda t: (t, 0)),
            pl.BlockSpec((_RT, _E), lambda t: (t, 0)),
            pl.BlockSpec((_RT, _H), lambda t: (t, 0)),
        ],
        out_shape=[
            jax.ShapeDtypeStruct((T, _E), jnp.float32),
            jax.ShapeDtypeStruct((T, _E), jnp.float32),
            jax.ShapeDtypeStruct((T, _H), jnp.bfloat16),
        ],
    )(hs, gate_w)

    out = pl.pallas_call(
        _expert_body,
        grid=(_E, _F // _FH, T // _TT),
        in_specs=[
            pl.BlockSpec((T, _H), lambda e, f, t: (0, 0)),
            pl.BlockSpec((T, _E), lambda e, f, t: (0, 0)),
            pl.BlockSpec((1, _H, _FH), lambda e, f, t: (e, 0, f)),
            pl.BlockSpec((1, _H, _FH), lambda e, f, t: (e, 0, f + _F // _FH)),
            pl.BlockSpec((1, _FH, _H), lambda e, f, t: (e, f, 0)),
        ],
        out_specs=pl.BlockSpec((T, _H), lambda e, f, t: (0, 0)),
        out_shape=jax.ShapeDtypeStruct((T, _H), jnp.float32),
        scratch_shapes=[
            pltpu.VMEM((_H, _FH), jnp.bfloat16),
            pltpu.VMEM((_H, _FH), jnp.bfloat16),
            pltpu.VMEM((_FH, _H), jnp.bfloat16),
        ],
        compiler_params=pltpu.CompilerParams(
            dimension_semantics=("arbitrary", "arbitrary", "arbitrary")),
    )(xb, wdense, gate_up_proj, gate_up_proj, down_proj)
    return out.reshape(_B, _S, _H), logits


# speedup vs baseline: 1.0013x; 1.0013x over previous
import jax
import jax.numpy as jnp
from jax.experimental import pallas as pl
from jax.experimental.pallas import tpu as pltpu

_B, _S, _H, _E, _F = 1, 2048, 2048, 8, 768
_FH = 384
_TT = 2048
_HT = 512
_RT = 512


def _router_body(hs_ref, gw_ref, logits_ref, wd_ref, xb_ref):
    x = hs_ref[...]
    xb_ref[...] = x.astype(jnp.bfloat16)
    logits = jax.lax.dot_general(
        x, gw_ref[...], (((1,), (1,)), ((), ())),
        preferred_element_type=jnp.float32)
    logits_ref[...] = logits
    p = jax.nn.softmax(logits, axis=-1)
    eio = jax.lax.broadcasted_iota(jnp.int32, p.shape, 1)
    m1 = jnp.max(p, axis=-1, keepdims=True)
    i1 = jnp.argmax(p, axis=-1)[:, None]
    oh1 = eio == i1
    p2 = jnp.where(oh1, -jnp.inf, p)
    m2 = jnp.max(p2, axis=-1, keepdims=True)
    i2 = jnp.argmax(p2, axis=-1)[:, None]
    oh2 = eio == i2
    wd_ref[...] = (jnp.where(oh1, m1, 0.0)
                   + jnp.where(oh2, m2, 0.0)) / (m1 + m2)


def _expert_body(xb_ref, wd_ref, wg_ref, wu_ref, w2_ref, out_ref,
                 wgb_ref, wub_ref, w2b_ref):
    e = pl.program_id(0)
    f = pl.program_id(1)
    t = pl.program_id(2)

    @pl.when(t == 0)
    def _():
        wgb_ref[...] = wg_ref[0].astype(jnp.bfloat16)
        wub_ref[...] = wu_ref[0].astype(jnp.bfloat16)
        w2b_ref[...] = w2_ref[0].astype(jnp.bfloat16)

    @pl.when(jnp.logical_and(jnp.logical_and(e == 0, f == 0), t == 0))
    def _():
        out_ref[...] = jnp.zeros_like(out_ref)
    eio = jax.lax.broadcasted_iota(jnp.int32, (_HT, _E), 1)
    wgb = wgb_ref[...]
    wub = wub_ref[...]
    w2b = w2b_ref[...]

    row_sl = [pl.ds(t * _TT + h * _HT, _HT) for h in range(_TT // _HT)]
    xs = [xb_ref[r, :] for r in row_sl]
    gs = [jnp.dot(x, wgb, preferred_element_type=jnp.float32) for x in xs]
    us = [jnp.dot(x, wub, preferred_element_type=jnp.float32) for x in xs]
    contribs = []
    for h, r in enumerate(row_sl):
        we = jnp.sum(jnp.where(eio == e, wd_ref[r, :], 0.0),
                     axis=-1, keepdims=True)
        g, u = gs[h], us[h]
        inter = (we * (u * (g * jax.nn.sigmoid(g)))).astype(jnp.bfloat16)
        contribs.append(
            jnp.dot(inter, w2b, preferred_element_type=jnp.float32))
    for h, r in enumerate(row_sl):
        out_ref[r, :] += contribs[h]


def kernel(hidden_states, gate_w, gate_up_proj, down_proj):
    T = _B * _S
    hs = hidden_states.reshape(T, _H)
    logits, wdense, xb = pl.pallas_call(
        _router_body,
        grid=(T // _RT,),
        in_specs=[
            pl.BlockSpec((_RT, _H), lambda t: (t, 0)),
            pl.BlockSpec((_E, _H), lambda t: (0, 0)),
        ],
        out_specs=[
            pl.BlockSpec((_RT, _E), lambda t: (t, 0)),
            pl.BlockSpec((_RT, _E), lambda t: (t, 0)),
            pl.BlockSpec((_RT, _H), lambda t: (t, 0)),
        ],
        out_shape=[
            jax.ShapeDtypeStruct((T, _E), jnp.float32),
            jax.ShapeDtypeStruct((T, _E), jnp.float32),
            jax.ShapeDtypeStruct((T, _H), jnp.bfloat16),
        ],
    )(hs, gate_w)

    out = pl.pallas_call(
        _expert_body,
        grid=(_E, _F // _FH, T // _TT),
        in_specs=[
            pl.BlockSpec((T, _H), lambda e, f, t: (0, 0)),
            pl.BlockSpec((T, _E), lambda e, f, t: (0, 0)),
            pl.BlockSpec((1, _H, _FH), lambda e, f, t: (e, 0, f)),
            pl.BlockSpec((1, _H, _FH), lambda e, f, t: (e, 0, f + _F // _FH)),
            pl.BlockSpec((1, _FH, _H), lambda e, f, t: (e, f, 0)),
        ],
        out_specs=pl.BlockSpec((T, _H), lambda e, f, t: (0, 0)),
        out_shape=jax.ShapeDtypeStruct((T, _H), jnp.float32),
        scratch_shapes=[
            pltpu.VMEM((_H, _FH), jnp.bfloat16),
            pltpu.VMEM((_H, _FH), jnp.bfloat16),
            pltpu.VMEM((_FH, _H), jnp.bfloat16),
        ],
        compiler_params=pltpu.CompilerParams(
            dimension_semantics=("arbitrary", "arbitrary", "arbitrary")),
    )(xb, wdense, gate_up_proj, gate_up_proj, down_proj)
    return out.reshape(_B, _S, _H), logits
